# baseline (speedup 1.0000x reference)
import functools

import jax
import jax.numpy as jnp
from jax import lax
from jax.experimental import pallas as pl
from jax.experimental.pallas import tpu as pltpu
from jax.experimental.pallas import tpu_sc as plsc

_B, _H, _W, _C = 128, 32, 32, 128
_HW = _H * _W
_P = 128
_LANES = 16
_NSUB = _B // _LANES


def _k1_body(x_ref, w_ref, b_ref, d1_ref, d2_ref):
    xm = x_ref[...].reshape(_B * _P, _C)
    d = jax.lax.dot_general(xm, w_ref[...], (((1,), (0,)), ((), ())),
                            preferred_element_type=jnp.float32)
    d1_ref[...] = d[:, 0].reshape(_B, _P) + b_ref[0, 0]
    d2_ref[...] = d[:, 1].reshape(_B, _P)


def _run_k1(xr, w12, b2):
    return pl.pallas_call(
        _k1_body,
        grid=(_HW // _P,),
        in_specs=[
            pl.BlockSpec((_B, _P, _C), lambda k: (0, k, 0)),
            pl.BlockSpec((_C, 2), lambda k: (0, 0)),
            pl.BlockSpec(memory_space=pltpu.SMEM),
        ],
        out_specs=[
            pl.BlockSpec((_B, _P), lambda k: (0, k)),
            pl.BlockSpec((_B, _P), lambda k: (0, k)),
        ],
        out_shape=[
            jax.ShapeDtypeStruct((_B, _HW), jnp.float32),
            jax.ShapeDtypeStruct((_B, _HW), jnp.float32),
        ],
    )(xr, w12, b2)


_MESH = plsc.VectorSubcoreMesh(core_axis_name="c", subcore_axis_name="s",
                               num_cores=1)


_ST = _HW + 16
_DUMMY = _LANES * _ST
_NSLOT = 6
_NCHUNK = 4
_CH = _HW // _NCHUNK
_CHW = _CH * _NSLOT * _LANES


@functools.partial(
    pl.kernel,
    out_type=jax.ShapeDtypeStruct((_B * _HW,), jnp.float32),
    mesh=_MESH,
    compiler_params=pltpu.CompilerParams(needs_layout_passes=False),
    scratch_types=[
        pltpu.VMEM((_LANES * _ST,), jnp.float32),
        pltpu.VMEM((_LANES * _ST + 8,), jnp.float32),
        pltpu.VMEM((_LANES * _ST,), jnp.float32),
        pltpu.VMEM((_CHW,), jnp.int32),
        pltpu.VMEM((_CHW,), jnp.int32),
        pltpu.VMEM((2 * _C,), jnp.float32),
        pltpu.VMEM((_C,), jnp.float32),
        pltpu.SemaphoreType.DMA,
        pltpu.SemaphoreType.DMA,
        pltpu.SemaphoreType.DMA,
    ],
)
def _k2(d1_hbm, d2_hbm, tab_hbm, wc_hbm, wy_hbm,
        y_hbm, d1v, vv, yout, tb0, tb1, wcv, wyv, sem0, sem1, semi):
    wid = lax.axis_index("s") + lax.axis_index("c")

    @pl.when(wid < _NSUB)
    def _():
        base = wid * _LANES
        tbs, sems = (tb0, tb1), (sem0, sem1)
        copies = [pltpu.async_copy(tab_hbm.at[pl.ds(c * _CHW, _CHW)],
                                   tbs[c % 2], sems[c % 2])
                  for c in range(2)]
        init = []
        for r in range(_LANES):
            src1 = d1_hbm.at[pl.ds((base + r) * _HW, _HW)]
            src2 = d2_hbm.at[pl.ds((base + r) * _HW, _HW)]
            init.append(pltpu.async_copy(
                src1, d1v.at[pl.ds(r * _ST, _HW)], semi))
            init.append(pltpu.async_copy(
                src2, vv.at[pl.ds(r * _ST, _HW)], semi))
        init.append(pltpu.async_copy(wc_hbm, wcv, semi))
        init.append(pltpu.async_copy(wy_hbm, wyv, semi))
        for cp in init:
            cp.wait()

        zero = jnp.zeros((_LANES,), jnp.float32)
        plsc.store_scatter(vv, [jnp.full((_LANES,), _DUMMY, jnp.int32)], zero)

        sacc = zero
        for j in range(_C // _LANES):
            sacc = sacc + (wcv[pl.ds(_C + j * _LANES, _LANES)]
                           * wyv[pl.ds(j * _LANES, _LANES)])
        s = sacc[0]
        for l in range(1, _LANES):
            s = s + sacc[l]
        svec = jnp.full((_LANES,), s, jnp.float32)

        for c in range(_NCHUNK):
            copies[c % 2].wait()
            if c + 1 < _NCHUNK:
                copies.append(
                    pltpu.async_copy(
                        tab_hbm.at[pl.ds((c + 1) * _CHW, _CHW)],
                        tbs[(c + 1) % 2], sems[(c + 1) % 2]))
            tb = tbs[c % 2]

            def load_rows(ii):
                b = ii * (_NSLOT * _LANES)
                return (tb[pl.ds(b, _LANES)],
                        tb[pl.ds(b + _LANES, _LANES)],
                        tb[pl.ds(b + 2 * _LANES, _LANES)],
                        tb[pl.ds(b + 3 * _LANES, _LANES)],
                        tb[pl.ds(b + 4 * _LANES, _LANES)],
                        tb[pl.ds(b + 5 * _LANES, _LANES)])

            def step(ii, carry):
                n0, n1, n2, n3, pcol, cbits = carry
                acc = (plsc.load_gather(vv, [n0])
                       + plsc.load_gather(vv, [n1])
                       + plsc.load_gather(vv, [n2])
                       + plsc.load_gather(vv, [n3]))
                crow = plsc.bitcast(cbits, jnp.float32)
                y = plsc.load_gather(d1v, [pcol]) + crow * acc
                nxt = load_rows(jnp.minimum(ii + 1, _CH - 1))
                plsc.addupdate_scatter(vv, [pcol], svec * y)
                plsc.store_scatter(yout, [pcol], y)
                return nxt

            lax.fori_loop(0, _CH, step, load_rows(0), unroll=8)

        out = [pltpu.async_copy(yout.at[pl.ds(r * _ST, _HW)],
                                y_hbm.at[pl.ds((base + r) * _HW, _HW)], semi)
               for r in range(_LANES)]
        for cp in out:
            cp.wait()


def _k3_body(x_ref, y_ref, w_ref, o_ref):
    yb = y_ref[...]
    wy = w_ref[0, :]
    o_ref[...] = x_ref[...] + yb[:, :, None] * wy[None, None, :]


def _run_k3(xr, y, wy2):
    return pl.pallas_call(
        _k3_body,
        grid=(_HW // _P,),
        in_specs=[
            pl.BlockSpec((_B, _P, _C), lambda k: (0, k, 0)),
            pl.BlockSpec((_B, _P), lambda k: (0, k)),
            pl.BlockSpec((1, _C), lambda k: (0, 0)),
        ],
        out_specs=pl.BlockSpec((_B, _P, _C), lambda k: (0, k, 0)),
        out_shape=jax.ShapeDtypeStruct((_B, _HW, _C), jnp.float32),
    )(xr, y, wy2)


def kernel(x, w_cell, b_cell, w_yvec, path, neigh_idx, neigh_valid):
    xr = x.reshape(_B, _HW, _C)
    w12 = w_cell.reshape(2, _C).T
    b2 = b_cell.reshape(1, 1)
    mask = neigh_valid > 0
    cntinv = 1.0 / jnp.maximum(jnp.sum(mask, axis=-1).astype(jnp.float32), 1.0)
    lanes = jnp.arange(_LANES, dtype=jnp.int32)
    laneoff = lanes[None, :] * _ST
    p32 = path.astype(jnp.int32)
    nflat = jnp.where(mask[:, :, None],
                      neigh_idx.astype(jnp.int32)[:, :, None] + laneoff[None],
                      _DUMMY)
    pflat = (p32[:, None] + laneoff)[:, None, :]
    cbits = jnp.tile(lax.bitcast_convert_type(cntinv, jnp.int32)[:, None],
                     (1, _LANES))[:, None, :]
    tab = jnp.concatenate([nflat, pflat, cbits], axis=1).reshape(-1)
    d1, d2 = _run_k1(xr, w12, b2)
    y = _k2(d1.reshape(-1), d2.reshape(-1), tab, w_cell, w_yvec)
    out = _run_k3(xr, y.reshape(_B, _HW), w_yvec.reshape(1, _C))
    return out.reshape(_B, _H, _W, _C)


# speedup vs baseline: 1.0806x; 1.0806x over previous
import functools

import jax
import jax.numpy as jnp
from jax import lax
from jax.experimental import pallas as pl
from jax.experimental.pallas import tpu as pltpu
from jax.experimental.pallas import tpu_sc as plsc

_B, _H, _W, _C = 128, 32, 32, 128
_HW = _H * _W
_P = 128
_LANES = 16
_NSUB = _B // _LANES


def _k1_body(x_ref, w_ref, b_ref, d1_ref, d2_ref):
    xm = x_ref[...].reshape(_B * _P, _C)
    d = jax.lax.dot_general(xm, w_ref[...], (((1,), (0,)), ((), ())),
                            preferred_element_type=jnp.float32)
    d1_ref[...] = d[:, 0].reshape(_B, _P) + b_ref[0, 0]
    d2_ref[...] = d[:, 1].reshape(_B, _P)


def _run_k1(xr, w12, b2):
    return pl.pallas_call(
        _k1_body,
        grid=(_HW // _P,),
        in_specs=[
            pl.BlockSpec((_B, _P, _C), lambda k: (0, k, 0)),
            pl.BlockSpec((_C, 2), lambda k: (0, 0)),
            pl.BlockSpec(memory_space=pltpu.SMEM),
        ],
        out_specs=[
            pl.BlockSpec((_B, _P), lambda k: (0, k)),
            pl.BlockSpec((_B, _P), lambda k: (0, k)),
        ],
        out_shape=[
            jax.ShapeDtypeStruct((_B, _HW), jnp.float32),
            jax.ShapeDtypeStruct((_B, _HW), jnp.float32),
        ],
    )(xr, w12, b2)


_MESH = plsc.VectorSubcoreMesh(core_axis_name="c", subcore_axis_name="s",
                               num_cores=1)


_ST = _HW + 8
_DUMMY = _LANES * _ST
_NSLOT = 6
_NCHUNK = 4
_CH = _HW // _NCHUNK
_CHW = _CH * _NSLOT * _LANES


@functools.partial(
    pl.kernel,
    out_type=jax.ShapeDtypeStruct((_B * _HW,), jnp.float32),
    mesh=_MESH,
    compiler_params=pltpu.CompilerParams(needs_layout_passes=False),
    scratch_types=[
        pltpu.VMEM((_LANES * _ST,), jnp.float32),
        pltpu.VMEM((_LANES * _ST + 8,), jnp.float32),
        pltpu.VMEM((_LANES * _ST,), jnp.float32),
        pltpu.VMEM((_CHW,), jnp.int32),
        pltpu.VMEM((_CHW,), jnp.int32),
        pltpu.VMEM((2 * _C,), jnp.float32),
        pltpu.VMEM((_C,), jnp.float32),
        pltpu.SemaphoreType.DMA,
        pltpu.SemaphoreType.DMA,
        pltpu.SemaphoreType.DMA,
    ],
)
def _k2(d1_hbm, d2_hbm, tab_hbm, wc_hbm, wy_hbm,
        y_hbm, d1v, vv, yout, tb0, tb1, wcv, wyv, sem0, sem1, semi):
    wid = lax.axis_index("s") + lax.axis_index("c")

    @pl.when(wid < _NSUB)
    def _():
        base = wid * _LANES
        tbs, sems = (tb0, tb1), (sem0, sem1)
        copies = [pltpu.async_copy(tab_hbm.at[pl.ds(c * _CHW, _CHW)],
                                   tbs[c % 2], sems[c % 2])
                  for c in range(2)]
        init = []
        for r in range(_LANES):
            src1 = d1_hbm.at[pl.ds((base + r) * _HW, _HW)]
            src2 = d2_hbm.at[pl.ds((base + r) * _HW, _HW)]
            init.append(pltpu.async_copy(
                src1, d1v.at[pl.ds(r * _ST, _HW)], semi))
            init.append(pltpu.async_copy(
                src2, vv.at[pl.ds(r * _ST, _HW)], semi))
        init.append(pltpu.async_copy(wc_hbm, wcv, semi))
        init.append(pltpu.async_copy(wy_hbm, wyv, semi))
        for cp in init:
            cp.wait()

        zero = jnp.zeros((_LANES,), jnp.float32)
        plsc.store_scatter(vv, [jnp.full((_LANES,), _DUMMY, jnp.int32)], zero)

        sacc = zero
        for j in range(_C // _LANES):
            sacc = sacc + (wcv[pl.ds(_C + j * _LANES, _LANES)]
                           * wyv[pl.ds(j * _LANES, _LANES)])
        s = sacc[0]
        for l in range(1, _LANES):
            s = s + sacc[l]
        svec = jnp.full((_LANES,), s, jnp.float32)

        for c in range(_NCHUNK):
            copies[c % 2].wait()
            if c + 1 < _NCHUNK:
                copies.append(
                    pltpu.async_copy(
                        tab_hbm.at[pl.ds((c + 1) * _CHW, _CHW)],
                        tbs[(c + 1) % 2], sems[(c + 1) % 2]))
            tb = tbs[c % 2]

            def load_rows(ii):
                b = ii * (_NSLOT * _LANES)
                return (tb[pl.ds(b, _LANES)],
                        tb[pl.ds(b + _LANES, _LANES)],
                        tb[pl.ds(b + 2 * _LANES, _LANES)],
                        tb[pl.ds(b + 3 * _LANES, _LANES)],
                        tb[pl.ds(b + 4 * _LANES, _LANES)],
                        tb[pl.ds(b + 5 * _LANES, _LANES)])

            def step(ii, carry):
                n0, n1, n2, n3, pcol, cbits = carry
                acc = (plsc.load_gather(vv, [n0])
                       + plsc.load_gather(vv, [n1])
                       + plsc.load_gather(vv, [n2])
                       + plsc.load_gather(vv, [n3]))
                crow = plsc.bitcast(cbits, jnp.float32)
                y = plsc.load_gather(d1v, [pcol]) + crow * acc
                nxt = load_rows(jnp.minimum(ii + 1, _CH - 1))
                plsc.addupdate_scatter(vv, [pcol], svec * y)
                plsc.store_scatter(yout, [pcol], y)
                return nxt

            lax.fori_loop(0, _CH, step, load_rows(0), unroll=16)

        out = [pltpu.async_copy(yout.at[pl.ds(r * _ST, _HW)],
                                y_hbm.at[pl.ds((base + r) * _HW, _HW)], semi)
               for r in range(_LANES)]
        for cp in out:
            cp.wait()


def _k3_body(x_ref, y_ref, w_ref, o_ref):
    yb = y_ref[...]
    wy = w_ref[0, :]
    o_ref[...] = x_ref[...] + yb[:, :, None] * wy[None, None, :]


def _run_k3(xr, y, wy2):
    return pl.pallas_call(
        _k3_body,
        grid=(_HW // _P,),
        in_specs=[
            pl.BlockSpec((_B, _P, _C), lambda k: (0, k, 0)),
            pl.BlockSpec((_B, _P), lambda k: (0, k)),
            pl.BlockSpec((1, _C), lambda k: (0, 0)),
        ],
        out_specs=pl.BlockSpec((_B, _P, _C), lambda k: (0, k, 0)),
        out_shape=jax.ShapeDtypeStruct((_B, _HW, _C), jnp.float32),
    )(xr, y, wy2)


def kernel(x, w_cell, b_cell, w_yvec, path, neigh_idx, neigh_valid):
    xr = x.reshape(_B, _HW, _C)
    w12 = w_cell.reshape(2, _C).T
    b2 = b_cell.reshape(1, 1)
    mask = neigh_valid > 0
    cntinv = 1.0 / jnp.maximum(jnp.sum(mask, axis=-1).astype(jnp.float32), 1.0)
    lanes = jnp.arange(_LANES, dtype=jnp.int32)
    laneoff = lanes[None, :] * _ST
    p32 = path.astype(jnp.int32)
    nflat = jnp.where(mask[:, :, None],
                      neigh_idx.astype(jnp.int32)[:, :, None] + laneoff[None],
                      _DUMMY)
    pflat = (p32[:, None] + laneoff)[:, None, :]
    cbits = jnp.tile(lax.bitcast_convert_type(cntinv, jnp.int32)[:, None],
                     (1, _LANES))[:, None, :]
    tab = jnp.concatenate([nflat, pflat, cbits], axis=1).reshape(-1)
    d1, d2 = _run_k1(xr, w12, b2)
    y = _k2(d1.reshape(-1), d2.reshape(-1), tab, w_cell, w_yvec)
    out = _run_k3(xr, y.reshape(_B, _HW), w_yvec.reshape(1, _C))
    return out.reshape(_B, _H, _W, _C)


# speedup vs baseline: 1.0919x; 1.0105x over previous
import functools

import jax
import jax.numpy as jnp
from jax import lax
from jax.experimental import pallas as pl
from jax.experimental.pallas import tpu as pltpu
from jax.experimental.pallas import tpu_sc as plsc

_B, _H, _W, _C = 128, 32, 32, 128
_HW = _H * _W
_P = 128
_LANES = 16
_NSUB = _B // _LANES


def _k1_body(x_ref, w_ref, b_ref, d1_ref, d2_ref):
    xm = x_ref[...].reshape(_B * _P, _C)
    d = jax.lax.dot_general(xm, w_ref[...], (((1,), (0,)), ((), ())),
                            preferred_element_type=jnp.float32)
    d1_ref[...] = d[:, 0].reshape(_B, _P) + b_ref[0, 0]
    d2_ref[...] = d[:, 1].reshape(_B, _P)


def _run_k1(xr, w12, b2):
    return pl.pallas_call(
        _k1_body,
        grid=(_HW // _P,),
        in_specs=[
            pl.BlockSpec((_B, _P, _C), lambda k: (0, k, 0)),
            pl.BlockSpec((_C, 2), lambda k: (0, 0)),
            pl.BlockSpec(memory_space=pltpu.SMEM),
        ],
        out_specs=[
            pl.BlockSpec((_B, _P), lambda k: (0, k)),
            pl.BlockSpec((_B, _P), lambda k: (0, k)),
        ],
        out_shape=[
            jax.ShapeDtypeStruct((_B, _HW), jnp.float32),
            jax.ShapeDtypeStruct((_B, _HW), jnp.float32),
        ],
    )(xr, w12, b2)


_MESH = plsc.VectorSubcoreMesh(core_axis_name="c", subcore_axis_name="s",
                               num_cores=1)


_ST = _HW + 8
_DUMMY = _LANES * _ST
_NSLOT = 6
_NCHUNK = 4
_CH = _HW // _NCHUNK
_CHW = _CH * _NSLOT * _LANES


@functools.partial(
    pl.kernel,
    out_type=jax.ShapeDtypeStruct((_B * _HW,), jnp.float32),
    mesh=_MESH,
    compiler_params=pltpu.CompilerParams(needs_layout_passes=False),
    scratch_types=[
        pltpu.VMEM((_LANES * _ST,), jnp.float32),
        pltpu.VMEM((_LANES * _ST + 8,), jnp.float32),
        pltpu.VMEM((_LANES * _ST,), jnp.float32),
        pltpu.VMEM((_CHW,), jnp.int32),
        pltpu.VMEM((_CHW,), jnp.int32),
        pltpu.VMEM((2 * _C,), jnp.float32),
        pltpu.VMEM((_C,), jnp.float32),
        pltpu.SemaphoreType.DMA,
        pltpu.SemaphoreType.DMA,
        pltpu.SemaphoreType.DMA,
    ],
)
def _k2(d1_hbm, d2_hbm, tab_hbm, wc_hbm, wy_hbm,
        y_hbm, d1v, vv, yout, tb0, tb1, wcv, wyv, sem0, sem1, semi):
    wid = lax.axis_index("s") + lax.axis_index("c")

    @pl.when(wid < _NSUB)
    def _():
        base = wid * _LANES
        tbs, sems = (tb0, tb1), (sem0, sem1)
        copies = [pltpu.async_copy(tab_hbm.at[pl.ds(c * _CHW, _CHW)],
                                   tbs[c % 2], sems[c % 2])
                  for c in range(2)]
        init = []
        for r in range(_LANES):
            src1 = d1_hbm.at[pl.ds((base + r) * _HW, _HW)]
            src2 = d2_hbm.at[pl.ds((base + r) * _HW, _HW)]
            init.append(pltpu.async_copy(
                src1, d1v.at[pl.ds(r * _ST, _HW)], semi))
            init.append(pltpu.async_copy(
                src2, vv.at[pl.ds(r * _ST, _HW)], semi))
        init.append(pltpu.async_copy(wc_hbm, wcv, semi))
        init.append(pltpu.async_copy(wy_hbm, wyv, semi))
        for cp in init:
            cp.wait()

        zero = jnp.zeros((_LANES,), jnp.float32)
        plsc.store_scatter(vv, [jnp.full((_LANES,), _DUMMY, jnp.int32)], zero)

        sacc = zero
        for j in range(_C // _LANES):
            sacc = sacc + (wcv[pl.ds(_C + j * _LANES, _LANES)]
                           * wyv[pl.ds(j * _LANES, _LANES)])
        s = sacc[0]
        for l in range(1, _LANES):
            s = s + sacc[l]
        svec = jnp.full((_LANES,), s, jnp.float32)

        for c in range(_NCHUNK):
            copies[c % 2].wait()
            if c + 1 < _NCHUNK:
                copies.append(
                    pltpu.async_copy(
                        tab_hbm.at[pl.ds((c + 1) * _CHW, _CHW)],
                        tbs[(c + 1) % 2], sems[(c + 1) % 2]))
            tb = tbs[c % 2]

            def load_rows(ii):
                b = ii * (_NSLOT * _LANES)
                return (tb[pl.ds(b, _LANES)],
                        tb[pl.ds(b + _LANES, _LANES)],
                        tb[pl.ds(b + 2 * _LANES, _LANES)],
                        tb[pl.ds(b + 3 * _LANES, _LANES)],
                        tb[pl.ds(b + 4 * _LANES, _LANES)],
                        tb[pl.ds(b + 5 * _LANES, _LANES)])

            def step(ii, carry):
                n0, n1, n2, n3, pcol, cbits = carry
                acc = ((plsc.load_gather(vv, [n0])
                        + plsc.load_gather(vv, [n1]))
                       + (plsc.load_gather(vv, [n2])
                          + plsc.load_gather(vv, [n3])))
                crow = plsc.bitcast(cbits, jnp.float32)
                y = plsc.load_gather(d1v, [pcol]) + crow * acc
                nxt = load_rows(jnp.minimum(ii + 1, _CH - 1))
                plsc.addupdate_scatter(vv, [pcol], svec * y)
                plsc.store_scatter(yout, [pcol], y)
                return nxt

            lax.fori_loop(0, _CH, step, load_rows(0), unroll=8)

        out = [pltpu.async_copy(yout.at[pl.ds(r * _ST, _HW)],
                                y_hbm.at[pl.ds((base + r) * _HW, _HW)], semi)
               for r in range(_LANES)]
        for cp in out:
            cp.wait()


def _k3_body(x_ref, y_ref, w_ref, o_ref):
    yb = y_ref[...]
    wy = w_ref[0, :]
    o_ref[...] = x_ref[...] + yb[:, :, None] * wy[None, None, :]


def _run_k3(xr, y, wy2):
    return pl.pallas_call(
        _k3_body,
        grid=(_HW // _P,),
        in_specs=[
            pl.BlockSpec((_B, _P, _C), lambda k: (0, k, 0)),
            pl.BlockSpec((_B, _P), lambda k: (0, k)),
            pl.BlockSpec((1, _C), lambda k: (0, 0)),
        ],
        out_specs=pl.BlockSpec((_B, _P, _C), lambda k: (0, k, 0)),
        out_shape=jax.ShapeDtypeStruct((_B, _HW, _C), jnp.float32),
    )(xr, y, wy2)


def kernel(x, w_cell, b_cell, w_yvec, path, neigh_idx, neigh_valid):
    xr = x.reshape(_B, _HW, _C)
    w12 = w_cell.reshape(2, _C).T
    b2 = b_cell.reshape(1, 1)
    mask = neigh_valid > 0
    cntinv = 1.0 / jnp.maximum(jnp.sum(mask, axis=-1).astype(jnp.float32), 1.0)
    lanes = jnp.arange(_LANES, dtype=jnp.int32)
    laneoff = lanes[None, :] * _ST
    p32 = path.astype(jnp.int32)
    nflat = jnp.where(mask[:, :, None],
                      neigh_idx.astype(jnp.int32)[:, :, None] + laneoff[None],
                      _DUMMY)
    pflat = (p32[:, None] + laneoff)[:, None, :]
    cbits = jnp.tile(lax.bitcast_convert_type(cntinv, jnp.int32)[:, None],
                     (1, _LANES))[:, None, :]
    tab = jnp.concatenate([nflat, pflat, cbits], axis=1).reshape(-1)
    d1, d2 = _run_k1(xr, w12, b2)
    y = _k2(d1.reshape(-1), d2.reshape(-1), tab, w_cell, w_yvec)
    out = _run_k3(xr, y.reshape(_B, _HW), w_yvec.reshape(1, _C))
    return out.reshape(_B, _H, _W, _C)
